# baseline (device time: 22477 ns/iter reference)
import jax
import jax.numpy as jnp
from jax import lax
from jax.experimental import pallas as pl
from jax.experimental.pallas import tpu as pltpu

N_DEV = 8
EPS = 1e-5
K = 8


def kernel(x, t_emb, W_scale, W_shift):
    b, s, c = x.shape
    c_global = c * N_DEV
    ck = s // K

    def body(x_hbm, t_ref, ws_ref, wsh_ref, out_hbm,
             xv, ov, stats_ref, in_sems, out_sems, send_sems, recv_sems):
        my_pos = lax.axis_index("i")

        in_copies = []
        for k in range(K):
            sl = pl.ds(k * ck, ck)
            cp = pltpu.make_async_copy(
                x_hbm.at[:, sl, :], xv.at[:, sl, :], in_sems.at[k])
            cp.start()
            in_copies.append(cp)

        scale = jnp.dot(t_ref[...], ws_ref[...],
                        preferred_element_type=jnp.float32)
        shift = jnp.dot(t_ref[...], wsh_ref[...],
                        preferred_element_type=jnp.float32)
        sc1 = 1.0 + scale

        for k in range(K):
            sl = slice(k * ck, (k + 1) * ck)
            in_copies[k].wait()
            xk = xv[:, sl, :]
            stats_ref[pl.ds(my_pos, 1), 0:b, sl] = \
                jnp.sum(xk, axis=-1)[None]
            stats_ref[pl.ds(my_pos, 1), b:2 * b, sl] = \
                jnp.sum(xk * xk, axis=-1)[None]

        barrier = pltpu.get_barrier_semaphore()
        for d in range(1, N_DEV):
            peer = (my_pos + d) % N_DEV
            pl.semaphore_signal(
                barrier, inc=1,
                device_id=(peer,), device_id_type=pl.DeviceIdType.MESH,
            )
        pl.semaphore_wait(barrier, N_DEV - 1)

        sends = []
        for d in range(1, N_DEV):
            peer = (my_pos + d) % N_DEV
            rdma = pltpu.make_async_remote_copy(
                src_ref=stats_ref.at[my_pos],
                dst_ref=stats_ref.at[my_pos],
                send_sem=send_sems.at[d],
                recv_sem=recv_sems.at[my_pos],
                device_id=(peer,),
                device_id_type=pl.DeviceIdType.MESH,
            )
            rdma.start()
            sends.append(rdma)

        for d in range(1, N_DEV):
            peer = (my_pos + d) % N_DEV
            recv = pltpu.make_async_remote_copy(
                src_ref=stats_ref.at[peer],
                dst_ref=stats_ref.at[peer],
                send_sem=send_sems.at[d],
                recv_sem=recv_sems.at[peer],
                device_id=(peer,),
                device_id_type=pl.DeviceIdType.MESH,
            )
            recv.wait_recv()

        tot = jnp.sum(stats_ref[...], axis=0)
        mean = tot[0:b] / c_global
        var = tot[b:2 * b] / c_global - mean * mean
        inv = lax.rsqrt(var + EPS)
        minv = -mean * inv

        out_copies = []
        for k in range(K):
            sl = slice(k * ck, (k + 1) * ck)
            xk = xv[:, sl, :]
            normed = xk * inv[:, sl, None] + minv[:, sl, None]
            outk = normed * sc1[:, None, :] + shift[:, None, :]
            ov[:, sl, :] = outk.astype(ov.dtype)
            cp = pltpu.make_async_copy(
                ov.at[:, sl, :], out_hbm.at[:, sl, :], out_sems.at[k])
            cp.start()
            out_copies.append(cp)

        for cp in out_copies:
            cp.wait()
        for rdma in sends:
            rdma.wait_send()

    return pl.pallas_call(
        body,
        out_shape=jax.ShapeDtypeStruct((b, s, c), jnp.bfloat16),
        in_specs=[
            pl.BlockSpec(memory_space=pl.ANY),
            pl.BlockSpec(memory_space=pltpu.VMEM),
            pl.BlockSpec(memory_space=pltpu.VMEM),
            pl.BlockSpec(memory_space=pltpu.VMEM),
        ],
        out_specs=pl.BlockSpec(memory_space=pl.ANY),
        scratch_shapes=[
            pltpu.VMEM((b, s, c), jnp.float32),
            pltpu.VMEM((b, s, c), jnp.bfloat16),
            pltpu.VMEM((N_DEV, 2 * b, s), jnp.float32),
            pltpu.SemaphoreType.DMA((K,)),
            pltpu.SemaphoreType.DMA((K,)),
            pltpu.SemaphoreType.DMA((N_DEV,)),
            pltpu.SemaphoreType.DMA((N_DEV,)),
        ],
        compiler_params=pltpu.CompilerParams(collective_id=0),
    )(x, t_emb, W_scale, W_shift)


# device time: 19126 ns/iter; 1.1752x vs baseline; 1.1752x over previous
import jax
import jax.numpy as jnp
from jax import lax
from jax.experimental import pallas as pl
from jax.experimental.pallas import tpu as pltpu

N_DEV = 8
EPS = 1e-5


def kernel(x, t_emb, W_scale, W_shift):
    b, s, c = x.shape
    c_global = c * N_DEV

    def body(x_ref, t_ref, ws_ref, wsh_ref, out_ref,
             stats_ref, send_sems, recv_sems):
        my_pos = lax.axis_index("i")

        barrier = pltpu.get_barrier_semaphore()
        for d in range(1, N_DEV):
            peer = (my_pos + d) % N_DEV
            pl.semaphore_signal(
                barrier, inc=1,
                device_id=(peer,), device_id_type=pl.DeviceIdType.MESH,
            )
        pl.semaphore_wait(barrier, N_DEV - 1)

        xf = x_ref[...].astype(jnp.float32)
        s1 = jnp.sum(xf, axis=-1)
        s2 = jnp.sum(xf * xf, axis=-1)
        local = jnp.concatenate([s1, s2], axis=0)
        stats_ref[pl.ds(my_pos, 1)] = local[None]

        sends = []
        for d in range(1, N_DEV):
            peer = (my_pos + d) % N_DEV
            rdma = pltpu.make_async_remote_copy(
                src_ref=stats_ref.at[my_pos],
                dst_ref=stats_ref.at[my_pos],
                send_sem=send_sems.at[d],
                recv_sem=recv_sems.at[my_pos],
                device_id=(peer,),
                device_id_type=pl.DeviceIdType.MESH,
            )
            rdma.start()
            sends.append(rdma)

        scale = jnp.dot(t_ref[...], ws_ref[...],
                        preferred_element_type=jnp.float32)
        shift = jnp.dot(t_ref[...], wsh_ref[...],
                        preferred_element_type=jnp.float32)
        sc1 = 1.0 + scale

        for d in range(1, N_DEV):
            peer = (my_pos + d) % N_DEV
            recv = pltpu.make_async_remote_copy(
                src_ref=stats_ref.at[peer],
                dst_ref=stats_ref.at[peer],
                send_sem=send_sems.at[d],
                recv_sem=recv_sems.at[peer],
                device_id=(peer,),
                device_id_type=pl.DeviceIdType.MESH,
            )
            recv.wait_recv()

        tot = jnp.sum(stats_ref[...], axis=0)
        mean = tot[0:b] / c_global
        var = tot[b:2 * b] / c_global - mean * mean
        inv = lax.rsqrt(var + EPS)
        minv = -mean * inv

        normed = xf * inv[:, :, None] + minv[:, :, None]
        out = normed * sc1[:, None, :] + shift[:, None, :]
        out_ref[...] = out.astype(out_ref.dtype)

        for rdma in sends:
            rdma.wait_send()

    return pl.pallas_call(
        body,
        out_shape=jax.ShapeDtypeStruct((b, s, c), jnp.bfloat16),
        in_specs=[
            pl.BlockSpec(memory_space=pltpu.VMEM),
            pl.BlockSpec(memory_space=pltpu.VMEM),
            pl.BlockSpec(memory_space=pltpu.VMEM),
            pl.BlockSpec(memory_space=pltpu.VMEM),
        ],
        out_specs=pl.BlockSpec(memory_space=pltpu.VMEM),
        scratch_shapes=[
            pltpu.VMEM((N_DEV, 2 * b, s), jnp.float32),
            pltpu.SemaphoreType.DMA((N_DEV,)),
            pltpu.SemaphoreType.DMA((N_DEV,)),
        ],
        compiler_params=pltpu.CompilerParams(collective_id=0),
    )(x, t_emb, W_scale, W_shift)
